# baseline (speedup 1.0000x reference)
import functools

import jax
import jax.numpy as jnp
import numpy as np
from jax.experimental import pallas as pl
from jax.experimental.pallas import tpu as pltpu

R = 32
D = 512
K = 6
M_REG = 8
FB_ALPHA = 0.1
FB_SCALE = 1.0 / np.sqrt(M_REG)


def _router_kernel(nbr_ref, h_ref, w_ref, coeff_ref, out_ref):
    r = pl.program_id(0)
    ys = []
    for k in range(K):
        idx = nbr_ref[r, k]
        h = h_ref[pl.ds(idx, 1), :]
        y = jax.lax.dot_general(
            h, w_ref[0, k],
            (((1,), (1,)), ((), ())),
            preferred_element_type=jnp.float32,
        )
        ys.append(y)
    Y = jnp.concatenate(ys, axis=0)
    c = coeff_ref[pl.ds(r, 1), :]
    out_ref[0] = jax.lax.dot_general(
        c, Y, (((1,), (0,)), ((), ())),
        preferred_element_type=jnp.float32,
    )


def kernel(H, reg_mask_prev, reg_coords, W_edge, W_reg, beta_cos, beta_sin, neighbors):
    delta = reg_coords[:, None, :] - jnp.take(reg_coords, neighbors, axis=0)
    S = jnp.einsum('rkd,md->rkm', delta, W_reg)
    b = (jnp.cos(S) * beta_cos + jnp.sin(S) * beta_sin).sum(-1) * FB_SCALE
    mask = jnp.take(reg_mask_prev, neighbors, axis=0).astype(H.dtype)
    coeff = (1.0 + FB_ALPHA * b) * mask

    grid = (R,)
    out = pl.pallas_call(
        _router_kernel,
        grid=grid,
        in_specs=[
            pl.BlockSpec(memory_space=pltpu.SMEM),
            pl.BlockSpec((R, D), lambda r: (0, 0)),
            pl.BlockSpec((1, K, D, D), lambda r: (r, 0, 0, 0)),
            pl.BlockSpec((R, K), lambda r: (0, 0)),
        ],
        out_specs=pl.BlockSpec((1, 1, D), lambda r: (r, 0, 0)),
        out_shape=jax.ShapeDtypeStruct((R, 1, D), jnp.float32),
        compiler_params=pltpu.CompilerParams(
            dimension_semantics=("arbitrary",),
        ),
    )(neighbors, H, W_edge, coeff)
    return out.reshape(R, D)
